# baseline (speedup 1.0000x reference)
import functools

import jax
import jax.numpy as jnp
from jax import lax
from jax.experimental import pallas as pl
from jax.experimental.pallas import tpu as pltpu
from jax.experimental.pallas import tpu_sc as plsc

USERS = 1000000
BATCH = 16384
D = 64
NC = 2
NS = 16
NW = NC * NS
BPW = BATCH // NW
NBLK = BPW // 16
MCHUNK = 128
NMCH = BPW // MCHUNK

_MESH = plsc.VectorSubcoreMesh(core_axis_name="c", subcore_axis_name="s")


@functools.partial(
    pl.kernel,
    mesh=_MESH,
    compiler_params=pltpu.CompilerParams(
        use_tc_tiling_on_sc=False, needs_layout_passes=False),
    out_type=(
        jax.ShapeDtypeStruct((BATCH, D), jnp.float32),
        jax.ShapeDtypeStruct((BATCH, D), jnp.float32),
    ),
    scratch_types=[
        pltpu.VMEM((BPW,), jnp.int32),
        pltpu.VMEM((2, 16, D, 16), jnp.float32),
        pltpu.VMEM((BPW, D), jnp.float32),
        pltpu.VMEM((BPW,), jnp.int32),
        pltpu.VMEM((BPW, D), jnp.float32),
        pltpu.SemaphoreType.DMA,
        pltpu.SemaphoreType.DMA,
    ],
)
def _sc_gather(user_tab_t, movie_table, user_ids, movie_ids,
               user_out, movie_gath,
               uids, sbuf, urows, midx, mrows, usem, msem):
    wid = lax.axis_index("s") * NC + lax.axis_index("c")
    base = wid * BPW
    pltpu.sync_copy(user_ids.at[pl.ds(base, BPW)], uids)
    pltpu.sync_copy(movie_ids.at[pl.ds(base, BPW)], midx)

    mcopies = []
    for j in range(NMCH):
        sl = pl.ds(j * MCHUNK, MCHUNK)
        mcopies.append(pltpu.async_copy(
            movie_table.at[midx.at[sl]], mrows.at[sl], msem))

    rows16 = jax.lax.iota(jnp.int32, 16)

    def fire(b, buf):
        uvec = uids[pl.ds(b * 16, 16)]
        cols = jnp.bitwise_and(uvec, jnp.int32(~15))
        for k in range(16):
            col = pl.multiple_of(cols[k], 16)
            pltpu.async_copy(
                user_tab_t.at[:, pl.ds(col, 16)],
                sbuf.at[buf, k], usem)

    def extract(b, buf):
        uvec = uids[pl.ds(b * 16, 16)]
        lanes = jnp.bitwise_and(uvec, jnp.int32(15))
        for k in range(16):
            pltpu.make_async_copy(
                user_tab_t.at[:, pl.ds(0, 16)], sbuf.at[buf, k], usem).wait()
        for k in range(16):
            lane = jnp.broadcast_to(lanes[k], (16,))
            i = b * 16 + k
            for r in range(D // 16):
                vals = plsc.load_gather(
                    sbuf.at[buf, k], [r * 16 + rows16, lane])
                urows[i, pl.ds(r * 16, 16)] = vals

    fire(0, 0)

    def blk(b, _):
        buf = lax.rem(b, 2)
        nbuf = lax.rem(b + 1, 2)

        @pl.when(b + 1 < NBLK)
        def _():
            fire(b + 1, nbuf)

        extract(b, buf)
        return ()

    lax.fori_loop(0, NBLK, blk, ())

    for c in mcopies:
        c.wait()
    pltpu.sync_copy(urows, user_out.at[pl.ds(base, BPW)])
    pltpu.sync_copy(mrows, movie_gath.at[pl.ds(base, BPW)])


def _tc_body(x_ref, w_ref, b_ref, g_ref, o_ref):
    o_ref[...] = (
        jnp.dot(x_ref[...], w_ref[...], preferred_element_type=jnp.float32)
        + b_ref[...] + g_ref[...]
    )


def kernel(movie_x, user_table, movie_table, W, b, user_node_id, movie_node_id):
    user_out, movie_gath = _sc_gather(
        user_table.T, movie_table, user_node_id, movie_node_id)
    BM = 2048
    movie_out = pl.pallas_call(
        _tc_body,
        grid=(BATCH // BM,),
        in_specs=[
            pl.BlockSpec((BM, 20), lambda i: (i, 0)),
            pl.BlockSpec((20, D), lambda i: (0, 0)),
            pl.BlockSpec((1, D), lambda i: (0, 0)),
            pl.BlockSpec((BM, D), lambda i: (i, 0)),
        ],
        out_specs=pl.BlockSpec((BM, D), lambda i: (i, 0)),
        out_shape=jax.ShapeDtypeStruct((BATCH, D), jnp.float32),
    )(movie_x, W, b.reshape(1, D), movie_gath)
    return (user_out, movie_out)
